# baseline (speedup 1.0000x reference)
import functools

import jax
import jax.numpy as jnp
from jax import lax
from jax.experimental import pallas as pl
from jax.experimental.pallas import tpu as pltpu
from jax.experimental.pallas import tpu_sc as plsc

_NUM_CORES = 2
_NUM_SUBCORES = 16
_NW = _NUM_CORES * _NUM_SUBCORES


def _sc_gather(emb_table, words_flat, batch, num_hist):
    d = emb_table.shape[1]
    b_rows = batch // _NW
    mesh = plsc.VectorSubcoreMesh(core_axis_name="c", subcore_axis_name="s")

    @functools.partial(
        pl.kernel,
        mesh=mesh,
        out_type=jax.ShapeDtypeStruct((batch, num_hist, d), jnp.float32),
        scratch_types=[
            pltpu.VMEM((b_rows, num_hist), jnp.int32),
            pltpu.VMEM((b_rows, num_hist, d), jnp.float32),
            pltpu.SemaphoreType.DMA,
        ],
        compiler_params=pltpu.CompilerParams(use_tc_tiling_on_sc=False),
    )
    def gather_kernel(table_hbm, idx_hbm, out_hbm, idx_v, rows_v, sem):
        wid = lax.axis_index("s") * _NUM_CORES + lax.axis_index("c")
        pltpu.sync_copy(idx_hbm.at[pl.ds(wid * b_rows, b_rows)], idx_v)
        copies = [
            pltpu.async_copy(table_hbm.at[idx_v.at[b]], rows_v.at[b], sem)
            for b in range(b_rows)
        ]
        for c in copies:
            c.wait()
        pltpu.sync_copy(rows_v, out_hbm.at[pl.ds(wid * b_rows, b_rows)])

    return gather_kernel(emb_table, words_flat)


def _h_body(feat_ref, w1_ref, b1_ref, h_ref):
    h = jnp.dot(feat_ref[...], w1_ref[...], preferred_element_type=jnp.float32)
    h_ref[...] = jnp.tanh(h + b1_ref[...])


def _logit_t_body(w2t_ref, ht_ref, b2_ref, out_ref):
    out_ref[...] = jnp.dot(w2t_ref[...], ht_ref[...],
                           preferred_element_type=jnp.float32) + b2_ref[...]


def kernel(words, emb_table, W1, b1, W2, b2):
    batch, num_hist = words.shape
    emb = emb_table.shape[1]
    feat_dim = num_hist * emb
    hid = W1.shape[1]
    vocab = W2.shape[1]

    feat = _sc_gather(emb_table, words.astype(jnp.int32), batch,
                      num_hist).reshape(batch, feat_dim)

    h = pl.pallas_call(
        _h_body,
        out_shape=jax.ShapeDtypeStruct((batch, hid), jnp.float32),
    )(feat, W1, b1.reshape(1, hid))
    ht = h.T

    w2t = W2.T
    b2col = b2.reshape(vocab, 1)

    vt = 2000
    grid = (vocab // vt,)
    out_t = pl.pallas_call(
        _logit_t_body,
        grid=grid,
        in_specs=[
            pl.BlockSpec((vt, hid), lambda i: (i, 0)),
            pl.BlockSpec((hid, batch), lambda i: (0, 0)),
            pl.BlockSpec((vt, 1), lambda i: (i, 0)),
        ],
        out_specs=pl.BlockSpec((vt, batch), lambda i: (i, 0)),
        out_shape=jax.ShapeDtypeStruct((vocab, batch), jnp.float32),
    )(w2t, ht, b2col)
    return out_t.T


# speedup vs baseline: 1.0716x; 1.0716x over previous
import functools

import jax
import jax.numpy as jnp
from jax import lax
from jax.experimental import pallas as pl
from jax.experimental.pallas import tpu as pltpu
from jax.experimental.pallas import tpu_sc as plsc

_NUM_CORES = 2
_NUM_SUBCORES = 16
_NW = _NUM_CORES * _NUM_SUBCORES


def _sc_gather(emb_table, words_flat, batch, num_hist):
    d = emb_table.shape[1]
    n_idx = batch * num_hist
    b_per_w = n_idx // _NW
    mesh = plsc.VectorSubcoreMesh(core_axis_name="c", subcore_axis_name="s")

    @functools.partial(
        pl.kernel,
        mesh=mesh,
        out_type=jax.ShapeDtypeStruct((n_idx, d), jnp.float32),
        scratch_types=[
            pltpu.VMEM((b_per_w,), jnp.int32),
            pltpu.VMEM((b_per_w, d), jnp.float32),
            pltpu.SemaphoreType.DMA,
        ],
        compiler_params=pltpu.CompilerParams(use_tc_tiling_on_sc=False),
    )
    def gather_kernel(table_hbm, idx_hbm, out_hbm, idx_v, rows_v, sem):
        wid = lax.axis_index("s") * _NUM_CORES + lax.axis_index("c")
        base = wid * b_per_w
        pltpu.sync_copy(idx_hbm.at[pl.ds(base, b_per_w)], idx_v)
        pltpu.async_copy(table_hbm.at[idx_v], rows_v, sem).wait()
        pltpu.sync_copy(rows_v, out_hbm.at[pl.ds(base, b_per_w)])

    return gather_kernel(emb_table, words_flat)


def _h_body(feat_ref, w1_ref, b1_ref, h_ref):
    h = jnp.dot(feat_ref[...], w1_ref[...], preferred_element_type=jnp.float32)
    h_ref[...] = jnp.tanh(h + b1_ref[...])


def _logit_t_body(w2t_ref, ht_ref, b2_ref, out_ref):
    out_ref[...] = jnp.dot(w2t_ref[...], ht_ref[...],
                           preferred_element_type=jnp.float32) + b2_ref[...]


def kernel(words, emb_table, W1, b1, W2, b2):
    batch, num_hist = words.shape
    emb = emb_table.shape[1]
    feat_dim = num_hist * emb
    hid = W1.shape[1]
    vocab = W2.shape[1]

    words_flat = words.reshape(-1).astype(jnp.int32)
    feat = _sc_gather(emb_table, words_flat, batch, num_hist).reshape(
        batch, feat_dim)

    h = pl.pallas_call(
        _h_body,
        out_shape=jax.ShapeDtypeStruct((batch, hid), jnp.float32),
    )(feat, W1, b1.reshape(1, hid))
    ht = h.T

    w2t = W2.T
    b2col = b2.reshape(vocab, 1)

    vt = 4000
    grid = (vocab // vt,)
    out_t = pl.pallas_call(
        _logit_t_body,
        grid=grid,
        in_specs=[
            pl.BlockSpec((vt, hid), lambda i: (i, 0)),
            pl.BlockSpec((hid, batch), lambda i: (0, 0)),
            pl.BlockSpec((vt, 1), lambda i: (i, 0)),
        ],
        out_specs=pl.BlockSpec((vt, batch), lambda i: (i, 0)),
        out_shape=jax.ShapeDtypeStruct((vocab, batch), jnp.float32),
    )(w2t, ht, b2col)
    return out_t.T


# speedup vs baseline: 1.0788x; 1.0067x over previous
import functools

import jax
import jax.numpy as jnp
from jax import lax
from jax.experimental import pallas as pl
from jax.experimental.pallas import tpu as pltpu
from jax.experimental.pallas import tpu_sc as plsc

_NUM_CORES = 2
_NUM_SUBCORES = 16
_NW = _NUM_CORES * _NUM_SUBCORES


def _sc_gather(emb_table, words_flat, batch, num_hist):
    d = emb_table.shape[1]
    n_idx = batch * num_hist
    b_per_w = n_idx // _NW
    mesh = plsc.VectorSubcoreMesh(core_axis_name="c", subcore_axis_name="s")

    @functools.partial(
        pl.kernel,
        mesh=mesh,
        out_type=jax.ShapeDtypeStruct((n_idx, d), jnp.float32),
        scratch_types=[
            pltpu.VMEM((b_per_w,), jnp.int32),
            pltpu.VMEM((b_per_w, d), jnp.float32),
            pltpu.SemaphoreType.DMA,
        ],
        compiler_params=pltpu.CompilerParams(use_tc_tiling_on_sc=False),
    )
    def gather_kernel(table_hbm, idx_hbm, out_hbm, idx_v, rows_v, sem):
        wid = lax.axis_index("s") * _NUM_CORES + lax.axis_index("c")
        base = wid * b_per_w
        pltpu.sync_copy(idx_hbm.at[pl.ds(base, b_per_w)], idx_v)
        pltpu.async_copy(table_hbm.at[idx_v], rows_v, sem).wait()
        pltpu.sync_copy(rows_v, out_hbm.at[pl.ds(base, b_per_w)])

    return gather_kernel(emb_table, words_flat)


def _ht_body(feat_ref, w1_ref, b1_ref, ht_ref):
    h = jnp.dot(feat_ref[...], w1_ref[...], preferred_element_type=jnp.float32)
    ht_ref[...] = jnp.tanh(h + b1_ref[...]).T


def _logit_t_body(w2t_ref, ht_ref, b2_ref, out_ref):
    out_ref[...] = jnp.dot(w2t_ref[...], ht_ref[...],
                           preferred_element_type=jnp.float32) + b2_ref[...]


def kernel(words, emb_table, W1, b1, W2, b2):
    batch, num_hist = words.shape
    emb = emb_table.shape[1]
    feat_dim = num_hist * emb
    hid = W1.shape[1]
    vocab = W2.shape[1]

    words_flat = words.reshape(-1).astype(jnp.int32)
    feat = _sc_gather(emb_table, words_flat, batch, num_hist).reshape(
        batch, feat_dim)

    ht = pl.pallas_call(
        _ht_body,
        out_shape=jax.ShapeDtypeStruct((hid, batch), jnp.float32),
    )(feat, W1, b1.reshape(1, hid))

    w2t = W2.T
    b2col = b2.reshape(vocab, 1)

    vt = 5000
    grid = (vocab // vt,)
    out_t = pl.pallas_call(
        _logit_t_body,
        grid=grid,
        in_specs=[
            pl.BlockSpec((vt, hid), lambda i: (i, 0)),
            pl.BlockSpec((hid, batch), lambda i: (0, 0)),
            pl.BlockSpec((vt, 1), lambda i: (i, 0)),
        ],
        out_specs=pl.BlockSpec((vt, batch), lambda i: (i, 0)),
        out_shape=jax.ShapeDtypeStruct((vocab, batch), jnp.float32),
    )(w2t, ht, b2col)
    return out_t.T
